# baseline (speedup 1.0000x reference)
import jax
import jax.numpy as jnp
from jax.experimental import pallas as pl

ROWS = 8192
COLS = 2048
BLOCK_ROWS = 256


def _mask_body(pos_ref, img_ref, out_ref):
    pos = pos_ref[:]
    col = jax.lax.broadcasted_iota(jnp.int32, (BLOCK_ROWS, COLS), 1)
    mask = col < pos[:, None]
    out_ref[:, :] = jnp.where(mask, img_ref[:, :], 0.0)


def kernel(images, position):
    grid = (ROWS // BLOCK_ROWS,)
    return pl.pallas_call(
        _mask_body,
        grid=grid,
        in_specs=[
            pl.BlockSpec((BLOCK_ROWS,), lambda i: (i,)),
            pl.BlockSpec((BLOCK_ROWS, COLS), lambda i: (i, 0)),
        ],
        out_specs=pl.BlockSpec((BLOCK_ROWS, COLS), lambda i: (i, 0)),
        out_shape=jax.ShapeDtypeStruct((ROWS, COLS), jnp.float32),
    )(position, images)
